# baseline (speedup 1.0000x reference)
import jax
import jax.numpy as jnp
from jax.experimental import pallas as pl
from jax.experimental.pallas import tpu as pltpu

_OUT_DOM = 64
_DIM_CONT = 128


def _body(ids_ref, x_ref, dvec_ref, w_ref, b_ref, trow_ref, out_ref):
    cin = x_ref.shape[1]
    s = x_ref.shape[2]
    out_ref[0, :cin, :] = x_ref[0]
    dvv = dvec_ref[0]
    dv = jnp.maximum(
        jnp.dot(dvv, w_ref[...], preferred_element_type=jnp.float32) + b_ref[...],
        0.0,
    )
    dv = dv + trow_ref[0]
    out_ref[0, cin:, :] = jnp.broadcast_to(dv.reshape(_OUT_DOM, 1), (_OUT_DOM, s))


def kernel(x, domain_ids, domain_vectors, W, b, table):
    bsz, cin, h, w = x.shape
    hw = h * w
    cout = cin + _OUT_DOM
    ns = 8
    s = hw // ns

    x3 = x.reshape(bsz, cin, hw)
    t3 = table.reshape(table.shape[0], 1, _OUT_DOM)
    dvec3 = domain_vectors.reshape(bsz, 1, _DIM_CONT)
    b2 = b.reshape(1, _OUT_DOM)

    out = pl.pallas_call(
        _body,
        grid_spec=pltpu.PrefetchScalarGridSpec(
            num_scalar_prefetch=1,
            grid=(bsz, ns),
            in_specs=[
                pl.BlockSpec((1, cin, s), lambda i, j, ids: (i, 0, j)),
                pl.BlockSpec((1, 1, _DIM_CONT), lambda i, j, ids: (i, 0, 0)),
                pl.BlockSpec((_DIM_CONT, _OUT_DOM), lambda i, j, ids: (0, 0)),
                pl.BlockSpec((1, _OUT_DOM), lambda i, j, ids: (0, 0)),
                pl.BlockSpec((1, 1, _OUT_DOM), lambda i, j, ids: (ids[i], 0, 0)),
            ],
            out_specs=pl.BlockSpec((1, cout, s), lambda i, j, ids: (i, 0, j)),
        ),
        out_shape=jax.ShapeDtypeStruct((bsz, cout, hw), x.dtype),
    )(domain_ids, x3, dvec3, W, b2, t3)
    return out.reshape(bsz, cout, h, w)
